# baseline (speedup 1.0000x reference)
import jax
import jax.numpy as jnp
from jax import lax
from jax.experimental import pallas as pl
from jax.experimental.pallas import tpu as pltpu
from jax.experimental.pallas import tpu_sc as plsc

NUM_CORES = 2
NUM_SUBCORES = 16
NW = NUM_CORES * NUM_SUBCORES
CHUNK = 256
NBUF = 4


def _emb_body(x_hbm, w_hbm, out_hbm, idx_v,
              b0, b1, b2, b3, g0, g1, g2, g3, s0, s1, s2, s3):
    nch = x_hbm.shape[1]
    bufs = (b0, b1, b2, b3)
    gsems = (g0, g1, g2, g3)
    ssems = (s0, s1, s2, s3)
    wid = lax.axis_index("s") * NUM_CORES + lax.axis_index("c")
    pltpu.sync_copy(x_hbm.at[wid], idx_v)
    for b in range(NBUF - 1):
        pltpu.async_copy(w_hbm.at[idx_v.at[b]], bufs[b], gsems[b])

    def body(i, carry):
        j0 = i * NBUF
        for b in range(NBUF):
            j = j0 + b
            pltpu.make_async_copy(w_hbm.at[idx_v.at[j]], bufs[b], gsems[b]).wait()
            pltpu.async_copy(bufs[b], out_hbm.at[wid, j], ssems[b])
            nb = (b + NBUF - 1) % NBUF
            jn = j + NBUF - 1

            @pl.when(jn < nch)
            def _(nb=nb, jn=jn):
                @pl.when(jn >= NBUF)
                def _():
                    pltpu.make_async_copy(
                        bufs[nb], out_hbm.at[wid, 0], ssems[nb]).wait()
                pltpu.async_copy(w_hbm.at[idx_v.at[jn]], bufs[nb], gsems[nb])
        return carry

    lax.fori_loop(0, nch // NBUF, body, 0)
    for b in range(NBUF):
        pltpu.make_async_copy(bufs[b], out_hbm.at[wid, 0], ssems[b]).wait()


def kernel(x, W):
    orig_shape = x.shape
    d = W.shape[1]
    b = x.size
    group = NW * CHUNK * NBUF
    b_pad = ((b + group - 1) // group) * group
    x_flat = x.reshape(-1).astype(jnp.int32)
    if b_pad != b:
        x_flat = jnp.pad(x_flat, (0, b_pad - b))
    nch = b_pad // (NW * CHUNK)
    x_r = x_flat.reshape(NW, nch, CHUNK)

    mesh = plsc.VectorSubcoreMesh(core_axis_name="c", subcore_axis_name="s")
    out = pl.kernel(
        _emb_body,
        out_type=jax.ShapeDtypeStruct((NW, nch, CHUNK, d), jnp.float32),
        mesh=mesh,
        scratch_types=(
            [pltpu.VMEM((nch, CHUNK), jnp.int32)]
            + [pltpu.VMEM((CHUNK, d), jnp.float32)] * NBUF
            + [pltpu.SemaphoreType.DMA] * (2 * NBUF)
        ),
        compiler_params=pltpu.CompilerParams(use_tc_tiling_on_sc=False),
    )(x_r, W)

    out = out.reshape(-1, d)
    if b_pad != b:
        out = out[:b]
    return out.reshape(*orig_shape, d)


# speedup vs baseline: 1.0718x; 1.0718x over previous
import jax
import jax.numpy as jnp
from jax import lax
from jax.experimental import pallas as pl
from jax.experimental.pallas import tpu as pltpu
from jax.experimental.pallas import tpu_sc as plsc

NUM_CORES = 2
NUM_SUBCORES = 16
NW = NUM_CORES * NUM_SUBCORES
CHUNK = 256
NBUF = 4


def _emb_body(x_hbm, w_hbm, out_hbm, idx_v,
              b0, b1, b2, b3, g0, g1, g2, g3, s0, s1, s2, s3):
    nch = x_hbm.shape[1]
    bufs = (b0, b1, b2, b3)
    gsems = (g0, g1, g2, g3)
    ssems = (s0, s1, s2, s3)
    wid = lax.axis_index("s") * NUM_CORES + lax.axis_index("c")
    pltpu.sync_copy(x_hbm.at[wid], idx_v)
    for b in range(NBUF - 1):
        pltpu.async_copy(w_hbm.at[idx_v.at[b]], bufs[b], gsems[b])

    def body(i, carry):
        j0 = i * NBUF
        for b in range(NBUF):
            j = j0 + b
            pltpu.make_async_copy(w_hbm.at[idx_v.at[j]], bufs[b], gsems[b]).wait()
            pltpu.async_copy(bufs[b], out_hbm.at[wid, j], ssems[b])
            nb = (b + NBUF - 1) % NBUF
            jn = j + NBUF - 1

            @pl.when(jn < nch)
            def _(nb=nb, jn=jn):
                @pl.when(jn >= NBUF)
                def _():
                    pltpu.make_async_copy(
                        bufs[nb], out_hbm.at[wid, 0], ssems[nb]).wait()
                pltpu.async_copy(w_hbm.at[idx_v.at[jn]], bufs[nb], gsems[nb])
        return carry

    lax.fori_loop(0, nch // NBUF, body, 0)
    for b in range(NBUF):
        pltpu.make_async_copy(bufs[b], out_hbm.at[wid, 0], ssems[b]).wait()


def kernel(x, W):
    orig_shape = x.shape
    d = W.shape[1]
    b = x.size
    group = NW * CHUNK * NBUF
    b_pad = ((b + group - 1) // group) * group
    x_flat = x.reshape(-1).astype(jnp.int32) * 2
    if b_pad != b:
        x_flat = jnp.pad(x_flat, (0, b_pad - b))
    nch = b_pad // (NW * CHUNK)
    x_r = x_flat.reshape(NW, nch, CHUNK)
    W = jnp.pad(W, ((0, 0), (0, d))).reshape(2 * W.shape[0], d)

    mesh = plsc.VectorSubcoreMesh(core_axis_name="c", subcore_axis_name="s")
    out = pl.kernel(
        _emb_body,
        out_type=jax.ShapeDtypeStruct((NW, nch, CHUNK, d), jnp.float32),
        mesh=mesh,
        scratch_types=(
            [pltpu.VMEM((nch, CHUNK), jnp.int32)]
            + [pltpu.VMEM((CHUNK, d), jnp.float32)] * NBUF
            + [pltpu.SemaphoreType.DMA] * (2 * NBUF)
        ),
        compiler_params=pltpu.CompilerParams(use_tc_tiling_on_sc=False),
    )(x_r, W)

    out = out.reshape(-1, d)
    if b_pad != b:
        out = out[:b]
    return out.reshape(*orig_shape, d)
